# baseline (speedup 1.0000x reference)
import functools

import jax
import jax.numpy as jnp
from jax.experimental import pallas as pl
from jax.experimental.pallas import tpu as pltpu

N = 8192
KNN = 32
HID = 64
ROWS = 256
WINW = 3072

_DN = (((1,), (0,)), ((), ()))


def _leaky(v):
    return jnp.where(v >= 0, v, 0.2 * v)


def _f32dot(a, b):
    return jax.lax.dot_general(a, b, _DN, preferred_element_type=jnp.float32)


def _select_gather_max(xb, xt, x, bcol, col0, wa, qi, width):
    rows = xb.shape[0]
    dot = _f32dot(xb.astype(jnp.bfloat16), xt.astype(jnp.bfloat16))
    sqj = jnp.sum(xt * xt, axis=0, keepdims=True)
    sqi = jnp.sum(xb * xb, axis=1, keepdims=True)
    d2 = (sqi + sqj) - 2.0 * dot
    work = jnp.where(bcol, jnp.float32(1e10), d2)
    colid = jax.lax.broadcasted_iota(jnp.int32, (rows, width), 1) + col0

    xhi = x.astype(jnp.bfloat16)
    xmid = (x - xhi.astype(jnp.float32)).astype(jnp.bfloat16)

    def step(work, acc):
        first = jnp.argmin(work, axis=1).astype(jnp.int32)[:, None] + col0
        eq = colid == first
        ef = eq.astype(jnp.bfloat16)
        xj = _f32dot(ef, xhi) + _f32dot(ef, xmid)
        dif = (xj - xb).astype(jnp.bfloat16)
        hk = _f32dot(dif, wa) + qi
        return (jnp.where(eq, jnp.float32(3e38), work),
                jnp.maximum(acc, hk))

    def body(_, carry):
        for _i in range(8):
            carry = step(*carry)
        return carry

    _, acc = jax.lax.fori_loop(
        0, KNN // 8, body,
        (work, jnp.full((rows, HID), -jnp.inf, jnp.float32)))
    return acc


def _edge_body(ws_ref, flag_ref, xb_ref, x_ref, xt_ref, wa_ref, wb_ref, b_ref,
               brow_ref, bcol_ref, out_ref):
    i = pl.program_id(0)
    xb = xb_ref[...]
    xib = xb.astype(jnp.bfloat16)
    qi = _f32dot(xib, wb_ref[...].astype(jnp.bfloat16)) + b_ref[...]
    wa = wa_ref[...].astype(jnp.bfloat16)
    brow = brow_ref[...]

    @pl.when(flag_ref[i] == 1)
    def _windowed():
        ws = pl.multiple_of(ws_ref[i], 512)
        xt = xt_ref[:, pl.ds(ws, WINW)]
        x = x_ref[pl.ds(ws, WINW), :]
        bcol = brow != bcol_ref[:, pl.ds(ws, WINW)]
        out_ref[...] = _leaky(
            _select_gather_max(xb, xt, x, bcol, ws, wa, qi, WINW))

    @pl.when(flag_ref[i] == 0)
    def _full():
        bcol = brow != bcol_ref[...]
        out_ref[...] = _leaky(
            _select_gather_max(xb, xt_ref[...], x_ref[...], bcol, 0, wa, qi, N))


def _edge_conv(x, brow, bcol, ws, flag, wa, wb, b):
    d = x.shape[1]
    xt = x.T
    return pl.pallas_call(
        _edge_body,
        grid=(N // ROWS,),
        in_specs=[
            pl.BlockSpec(memory_space=pltpu.SMEM),
            pl.BlockSpec(memory_space=pltpu.SMEM),
            pl.BlockSpec((ROWS, d), lambda i: (i, 0)),
            pl.BlockSpec((N, d), lambda i: (0, 0)),
            pl.BlockSpec((d, N), lambda i: (0, 0)),
            pl.BlockSpec((d, HID), lambda i: (0, 0)),
            pl.BlockSpec((d, HID), lambda i: (0, 0)),
            pl.BlockSpec((1, HID), lambda i: (0, 0)),
            pl.BlockSpec((ROWS, 1), lambda i: (i, 0)),
            pl.BlockSpec((1, N), lambda i: (0, 0)),
        ],
        out_specs=pl.BlockSpec((ROWS, HID), lambda i: (i, 0)),
        out_shape=jax.ShapeDtypeStruct((N, HID), jnp.float32),
    )(ws, flag, x, x, xt, wa, wb, b.reshape(1, HID), brow, bcol)


def _final_body(x1_ref, x2_ref, x3_ref, w1_ref, w2_ref, w3_ref, b_ref, out_ref):
    h = (_f32dot(x1_ref[...].astype(jnp.bfloat16), w1_ref[...])
         + _f32dot(x2_ref[...].astype(jnp.bfloat16), w2_ref[...])
         + _f32dot(x3_ref[...].astype(jnp.bfloat16), w3_ref[...])
         + b_ref[...])
    out_ref[...] = _leaky(h)


def _final(x1, x2, x3, wf, bf):
    emb = wf.shape[1]
    blk = 1024
    wf16 = wf.astype(jnp.bfloat16)
    return pl.pallas_call(
        _final_body,
        grid=(N // blk,),
        in_specs=[
            pl.BlockSpec((blk, HID), lambda i: (i, 0)),
            pl.BlockSpec((blk, HID), lambda i: (i, 0)),
            pl.BlockSpec((blk, HID), lambda i: (i, 0)),
            pl.BlockSpec((HID, emb), lambda i: (0, 0)),
            pl.BlockSpec((HID, emb), lambda i: (0, 0)),
            pl.BlockSpec((HID, emb), lambda i: (0, 0)),
            pl.BlockSpec((1, emb), lambda i: (0, 0)),
        ],
        out_specs=pl.BlockSpec((blk, emb), lambda i: (i, 0)),
        out_shape=jax.ShapeDtypeStruct((N, emb), jnp.float32),
    )(x1, x2, x3, wf16[:HID], wf16[HID:2 * HID], wf16[2 * HID:],
      bf.reshape(1, emb))


def kernel(xyz, features, batch, W1, b1, W2, b2, W3, b3, Wf, bf):
    b32 = batch.astype(jnp.int32)
    brow = b32.reshape(N, 1)
    bcol = b32.reshape(1, N)

    cids = jnp.arange(4, dtype=jnp.int32)
    starts = jnp.searchsorted(b32, cids, side="left").astype(jnp.int32)
    ends = jnp.searchsorted(b32, cids, side="right").astype(jnp.int32)
    any_small = jnp.any((ends - starts) < KNN)
    bfirst = b32[::ROWS]
    blast = b32[ROWS - 1::ROWS]
    ws = (starts[bfirst] // 512) * 512
    ws = jnp.minimum(ws, N - WINW)
    we = ends[blast]
    flag = ((we - ws <= WINW) & ~any_small).astype(jnp.int32)
    ws = ws.astype(jnp.int32)

    f = jnp.concatenate([features, xyz], axis=-1)
    d0 = f.shape[1]
    x1 = _edge_conv(f, brow, bcol, ws, flag, W1[:d0], W1[d0:], b1)
    x2 = _edge_conv(x1, brow, bcol, ws, flag, W2[:HID], W2[HID:], b2)
    x3 = _edge_conv(x2, brow, bcol, ws, flag, W3[:HID], W3[HID:], b3)
    return _final(x1, x2, x3, Wf, bf)


# speedup vs baseline: 1.2081x; 1.2081x over previous
import functools

import jax
import jax.numpy as jnp
from jax.experimental import pallas as pl
from jax.experimental.pallas import tpu as pltpu

N = 8192
KNN = 32
HID = 64
ROWS = 256
WINW = 2560

_DN = (((1,), (0,)), ((), ()))


def _leaky(v):
    return jnp.where(v >= 0, v, 0.2 * v)


def _f32dot(a, b):
    return jax.lax.dot_general(a, b, _DN, preferred_element_type=jnp.float32)


def _select_gather_max(xb, xt, x, bcol, col0, wa, qi, width):
    rows = xb.shape[0]
    dot = _f32dot(xb.astype(jnp.bfloat16), xt.astype(jnp.bfloat16))
    sqj = jnp.sum(xt * xt, axis=0, keepdims=True)
    sqi = jnp.sum(xb * xb, axis=1, keepdims=True)
    d2 = (sqi + sqj) - 2.0 * dot
    work = jnp.where(bcol, jnp.float32(1e10), d2)
    colid = jax.lax.broadcasted_iota(jnp.int32, (rows, width), 1) + col0

    xhi = x.astype(jnp.bfloat16)
    xmid = (x - xhi.astype(jnp.float32)).astype(jnp.bfloat16)

    def step(work, acc):
        first = jnp.argmin(work, axis=1).astype(jnp.int32)[:, None] + col0
        eq = colid == first
        ef = eq.astype(jnp.bfloat16)
        xj = _f32dot(ef, xhi) + _f32dot(ef, xmid)
        dif = (xj - xb).astype(jnp.bfloat16)
        hk = _f32dot(dif, wa) + qi
        return (jnp.where(eq, jnp.float32(3e38), work),
                jnp.maximum(acc, hk))

    def body(_, carry):
        for _i in range(4):
            carry = step(*carry)
        return carry

    _, acc = jax.lax.fori_loop(
        0, KNN // 4, body,
        (work, jnp.full((rows, HID), -jnp.inf, jnp.float32)))
    return acc


def _edge_body(ws_ref, flag_ref, xb_ref, x_ref, xt_ref, wa_ref, wb_ref, b_ref,
               brow_ref, bcol_ref, out_ref):
    i = pl.program_id(0)
    xb = xb_ref[...]
    xib = xb.astype(jnp.bfloat16)
    qi = _f32dot(xib, wb_ref[...].astype(jnp.bfloat16)) + b_ref[...]
    wa = wa_ref[...].astype(jnp.bfloat16)
    brow = brow_ref[...]

    @pl.when(flag_ref[i] == 1)
    def _windowed():
        ws = pl.multiple_of(ws_ref[i], 512)
        xt = xt_ref[:, pl.ds(ws, WINW)]
        x = x_ref[pl.ds(ws, WINW), :]
        bcol = brow != bcol_ref[:, pl.ds(ws, WINW)]
        out_ref[...] = _leaky(
            _select_gather_max(xb, xt, x, bcol, ws, wa, qi, WINW))

    @pl.when(flag_ref[i] == 0)
    def _full():
        bcol = brow != bcol_ref[...]
        out_ref[...] = _leaky(
            _select_gather_max(xb, xt_ref[...], x_ref[...], bcol, 0, wa, qi, N))


def _edge_conv(x, brow, bcol, ws, flag, wa, wb, b):
    d = x.shape[1]
    xt = x.T
    return pl.pallas_call(
        _edge_body,
        grid=(N // ROWS,),
        in_specs=[
            pl.BlockSpec(memory_space=pltpu.SMEM),
            pl.BlockSpec(memory_space=pltpu.SMEM),
            pl.BlockSpec((ROWS, d), lambda i: (i, 0)),
            pl.BlockSpec((N, d), lambda i: (0, 0)),
            pl.BlockSpec((d, N), lambda i: (0, 0)),
            pl.BlockSpec((d, HID), lambda i: (0, 0)),
            pl.BlockSpec((d, HID), lambda i: (0, 0)),
            pl.BlockSpec((1, HID), lambda i: (0, 0)),
            pl.BlockSpec((ROWS, 1), lambda i: (i, 0)),
            pl.BlockSpec((1, N), lambda i: (0, 0)),
        ],
        out_specs=pl.BlockSpec((ROWS, HID), lambda i: (i, 0)),
        out_shape=jax.ShapeDtypeStruct((N, HID), jnp.float32),
    )(ws, flag, x, x, xt, wa, wb, b.reshape(1, HID), brow, bcol)


def _final_body(x1_ref, x2_ref, x3_ref, w1_ref, w2_ref, w3_ref, b_ref, out_ref):
    h = (_f32dot(x1_ref[...].astype(jnp.bfloat16), w1_ref[...])
         + _f32dot(x2_ref[...].astype(jnp.bfloat16), w2_ref[...])
         + _f32dot(x3_ref[...].astype(jnp.bfloat16), w3_ref[...])
         + b_ref[...])
    out_ref[...] = _leaky(h)


def _final(x1, x2, x3, wf, bf):
    emb = wf.shape[1]
    blk = 1024
    wf16 = wf.astype(jnp.bfloat16)
    return pl.pallas_call(
        _final_body,
        grid=(N // blk,),
        in_specs=[
            pl.BlockSpec((blk, HID), lambda i: (i, 0)),
            pl.BlockSpec((blk, HID), lambda i: (i, 0)),
            pl.BlockSpec((blk, HID), lambda i: (i, 0)),
            pl.BlockSpec((HID, emb), lambda i: (0, 0)),
            pl.BlockSpec((HID, emb), lambda i: (0, 0)),
            pl.BlockSpec((HID, emb), lambda i: (0, 0)),
            pl.BlockSpec((1, emb), lambda i: (0, 0)),
        ],
        out_specs=pl.BlockSpec((blk, emb), lambda i: (i, 0)),
        out_shape=jax.ShapeDtypeStruct((N, emb), jnp.float32),
    )(x1, x2, x3, wf16[:HID], wf16[HID:2 * HID], wf16[2 * HID:],
      bf.reshape(1, emb))


def kernel(xyz, features, batch, W1, b1, W2, b2, W3, b3, Wf, bf):
    b32 = batch.astype(jnp.int32)
    brow = b32.reshape(N, 1)
    bcol = b32.reshape(1, N)

    cids = jnp.arange(4, dtype=jnp.int32)
    starts = jnp.searchsorted(b32, cids, side="left").astype(jnp.int32)
    ends = jnp.searchsorted(b32, cids, side="right").astype(jnp.int32)
    any_small = jnp.any((ends - starts) < KNN)
    bfirst = b32[::ROWS]
    blast = b32[ROWS - 1::ROWS]
    ws = (starts[bfirst] // 512) * 512
    ws = jnp.minimum(ws, N - WINW)
    we = ends[blast]
    flag = ((we - ws <= WINW) & ~any_small).astype(jnp.int32)
    ws = ws.astype(jnp.int32)

    f = jnp.concatenate([features, xyz], axis=-1)
    d0 = f.shape[1]
    x1 = _edge_conv(f, brow, bcol, ws, flag, W1[:d0], W1[d0:], b1)
    x2 = _edge_conv(x1, brow, bcol, ws, flag, W2[:HID], W2[HID:], b2)
    x3 = _edge_conv(x2, brow, bcol, ws, flag, W3[:HID], W3[HID:], b3)
    return _final(x1, x2, x3, Wf, bf)
